# Initial kernel scaffold; baseline (speedup 1.0000x reference)
import jax
import jax.numpy as jnp
from jax.experimental import pallas as pl


def kernel(encoded_outs, encoded_lens, embed, W_ih, W_hh, b_lstm, W_enc, W_pred, b_joint, W_out, b_out):
    raise NotImplementedError("write your pallas kernel here")


# speedup vs baseline: 4.9549x; 4.9549x over previous
import jax
import jax.numpy as jnp
from jax.experimental import pallas as pl
from jax.experimental.pallas import tpu as pltpu

_B = 16
_T = 512
_DE = 512
_DP = 320
_DJ = 320
_V = 1024
_BLANK = 0
_TCH = 128


def _decode_step(enc_ref, lens_ref, embed_ref,
                 wii_ref, wif_ref, wig_ref, wio_ref,
                 whi_ref, whf_ref, whg_ref, who_ref,
                 bli_ref, blf_ref, blg_ref, blo_ref,
                 wenc_ref, wpred_ref, bj_ref, wout_ref, bout_ref,
                 lab_ref, sc_ref,
                 h_ref, c_ref, lbl_ref):
    t = pl.program_id(0)

    @pl.when(t == 0)
    def _():
        h_ref[...] = jnp.zeros_like(h_ref)
        c_ref[...] = jnp.zeros_like(c_ref)
        lbl_ref[...] = jnp.full_like(lbl_ref, _BLANK)

    f32 = jnp.float32
    h = h_ref[...]
    c = c_ref[...]
    lbl = lbl_ref[...][:, :1]

    iota_v = jax.lax.broadcasted_iota(jnp.int32, (_B, _V), 1)
    onehot = (iota_v == lbl).astype(f32)
    emb = jnp.dot(onehot, embed_ref[...], preferred_element_type=f32)

    def gate(wi_ref, wh_ref, b_ref):
        return (jnp.dot(emb, wi_ref[...], preferred_element_type=f32)
                + jnp.dot(h, wh_ref[...], preferred_element_type=f32)
                + b_ref[...])

    g_i = gate(wii_ref, whi_ref, bli_ref)
    g_f = gate(wif_ref, whf_ref, blf_ref)
    g_g = gate(wig_ref, whg_ref, blg_ref)
    g_o = gate(wio_ref, who_ref, blo_ref)
    c_new = jax.nn.sigmoid(g_f) * c + jax.nn.sigmoid(g_i) * jnp.tanh(g_g)
    h_new = jax.nn.sigmoid(g_o) * jnp.tanh(c_new)

    enc_t = enc_ref[0]
    pre = (jnp.dot(enc_t, wenc_ref[...], preferred_element_type=f32)
           + jnp.dot(h_new, wpred_ref[...], preferred_element_type=f32)
           + bj_ref[...])
    logits = (jnp.dot(jnp.tanh(pre), wout_ref[...], preferred_element_type=f32)
              + bout_ref[...])

    m = jnp.max(logits, axis=1, keepdims=True)
    sym = jnp.min(jnp.where(logits == m, iota_v, _V), axis=1, keepdims=True)
    score = -jnp.log(jnp.sum(jnp.exp(logits - m), axis=1, keepdims=True))

    blank = jnp.logical_or(sym == _BLANK, t >= lens_ref[...][:, :1])
    h_ref[...] = jnp.where(blank, h, h_new)
    c_ref[...] = jnp.where(blank, c, c_new)
    new_lbl = jnp.where(blank, lbl, sym)
    lbl_ref[...] = jnp.broadcast_to(new_lbl, lbl_ref.shape)
    emit = jnp.where(blank, _BLANK, sym)

    col = jax.lax.rem(t, _TCH)
    colmask = jax.lax.broadcasted_iota(jnp.int32, (_B, _TCH), 1) == col
    lab_ref[...] = jnp.where(colmask, jnp.broadcast_to(emit, (_B, _TCH)),
                             lab_ref[...])
    sc_ref[...] = jnp.where(colmask, jnp.broadcast_to(score, (_B, _TCH)),
                            sc_ref[...])


def _full(shape):
    return pl.BlockSpec(shape, lambda t: (0,) * len(shape))


@jax.jit
def kernel(encoded_outs, encoded_lens, embed, W_ih, W_hh, b_lstm,
           W_enc, W_pred, b_joint, W_out, b_out):
    enc_tbd = jnp.transpose(encoded_outs, (1, 0, 2))
    lens_b = jnp.broadcast_to(encoded_lens.astype(jnp.int32)[:, None],
                              (_B, 128))
    wih = [W_ih[:, k * _DP:(k + 1) * _DP] for k in range(4)]
    whh = [W_hh[:, k * _DP:(k + 1) * _DP] for k in range(4)]
    bls = [b_lstm[None, k * _DP:(k + 1) * _DP] for k in range(4)]

    labels, scores = pl.pallas_call(
        _decode_step,
        grid=(_T,),
        in_specs=[
            pl.BlockSpec((1, _B, _DE), lambda t: (t, 0, 0)),
            _full((_B, 128)),
            _full((_V, _DP)),
            _full((_DP, _DP)), _full((_DP, _DP)),
            _full((_DP, _DP)), _full((_DP, _DP)),
            _full((_DP, _DP)), _full((_DP, _DP)),
            _full((_DP, _DP)), _full((_DP, _DP)),
            _full((1, _DP)), _full((1, _DP)),
            _full((1, _DP)), _full((1, _DP)),
            _full((_DE, _DJ)),
            _full((_DP, _DJ)),
            _full((1, _DJ)),
            _full((_DJ, _V)),
            _full((1, _V)),
        ],
        out_specs=[
            pl.BlockSpec((_B, _TCH), lambda t: (0, t // _TCH)),
            pl.BlockSpec((_B, _TCH), lambda t: (0, t // _TCH)),
        ],
        out_shape=[
            jax.ShapeDtypeStruct((_B, _T), jnp.int32),
            jax.ShapeDtypeStruct((_B, _T), jnp.float32),
        ],
        scratch_shapes=[
            pltpu.VMEM((_B, _DP), jnp.float32),
            pltpu.VMEM((_B, _DP), jnp.float32),
            pltpu.VMEM((_B, 128), jnp.int32),
        ],
        compiler_params=pltpu.CompilerParams(
            dimension_semantics=("arbitrary",)),
    )(enc_tbd, lens_b, embed,
      wih[0], wih[1], wih[2], wih[3],
      whh[0], whh[1], whh[2], whh[3],
      bls[0], bls[1], bls[2], bls[3],
      W_enc, W_pred, b_joint[None, :], W_out, b_out[None, :])
    return labels, scores


# speedup vs baseline: 5.2900x; 1.0676x over previous
import jax
import jax.numpy as jnp
from jax.experimental import pallas as pl
from jax.experimental.pallas import tpu as pltpu

_B = 16
_T = 512
_DE = 512
_DP = 320
_DJ = 320
_V = 1024
_BLANK = 0
_TCH = 128
_MB = 1024


def _proj_kernel(enc_ref, wenc_ref, bj_ref, out_ref):
    out_ref[...] = (jnp.dot(enc_ref[...], wenc_ref[...],
                            preferred_element_type=jnp.float32)
                    + bj_ref[...])


def _decode_kernel(encp_ref, lens_ref, embed_ref,
                   wii_ref, wif_ref, wig_ref, wio_ref,
                   whi_ref, whf_ref, whg_ref, who_ref,
                   bli_ref, blf_ref, blg_ref, blo_ref,
                   wpred_ref, wout_ref, bout_ref,
                   lab_ref, sc_ref):
    f32 = jnp.float32
    iota_v = jax.lax.broadcasted_iota(jnp.int32, (_B, _V), 1)
    iota_c = jax.lax.broadcasted_iota(jnp.int32, (_B, _TCH), 1)
    lens = lens_ref[...][:, :1]

    def step(chunk):
        def body(tt, carry):
            h, c, lbl, labacc, scacc = carry
            t = chunk * _TCH + tt

            onehot = (iota_v == lbl).astype(f32)
            emb = jnp.dot(onehot, embed_ref[...],
                          preferred_element_type=f32)

            def gate(wi_ref, wh_ref, b_ref):
                return (jnp.dot(emb, wi_ref[...], preferred_element_type=f32)
                        + jnp.dot(h, wh_ref[...], preferred_element_type=f32)
                        + b_ref[...])

            g_i = gate(wii_ref, whi_ref, bli_ref)
            g_f = gate(wif_ref, whf_ref, blf_ref)
            g_g = gate(wig_ref, whg_ref, blg_ref)
            g_o = gate(wio_ref, who_ref, blo_ref)
            c_new = (jax.nn.sigmoid(g_f) * c
                     + jax.nn.sigmoid(g_i) * jnp.tanh(g_g))
            h_new = jax.nn.sigmoid(g_o) * jnp.tanh(c_new)

            pre = encp_ref[t] + jnp.dot(h_new, wpred_ref[...],
                                        preferred_element_type=f32)
            logits = (jnp.dot(jnp.tanh(pre), wout_ref[...],
                              preferred_element_type=f32)
                      + bout_ref[...])

            m = jnp.max(logits, axis=1, keepdims=True)
            sym = jnp.min(jnp.where(logits == m, iota_v, _V),
                          axis=1, keepdims=True)
            score = -jnp.log(jnp.sum(jnp.exp(logits - m),
                                     axis=1, keepdims=True))

            blank = jnp.logical_or(sym == _BLANK, t >= lens)
            h = jnp.where(blank, h, h_new)
            c = jnp.where(blank, c, c_new)
            lbl = jnp.where(blank, lbl, sym)
            emit = jnp.where(blank, _BLANK, sym)

            colmask = iota_c == tt
            labacc = jnp.where(colmask,
                               jnp.broadcast_to(emit, (_B, _TCH)), labacc)
            scacc = jnp.where(colmask,
                              jnp.broadcast_to(score, (_B, _TCH)), scacc)
            return h, c, lbl, labacc, scacc
        return body

    h = jnp.zeros((_B, _DP), f32)
    c = jnp.zeros((_B, _DP), f32)
    lbl = jnp.full((_B, 1), _BLANK, jnp.int32)
    for chunk in range(_T // _TCH):
        init = (h, c, lbl,
                jnp.zeros((_B, _TCH), jnp.int32),
                jnp.zeros((_B, _TCH), f32))
        h, c, lbl, labacc, scacc = jax.lax.fori_loop(
            0, _TCH, step(chunk), init)
        lab_ref[:, chunk * _TCH:(chunk + 1) * _TCH] = labacc
        sc_ref[:, chunk * _TCH:(chunk + 1) * _TCH] = scacc


def _full(shape):
    return pl.BlockSpec(shape, lambda i: (0,) * len(shape))


@jax.jit
def kernel(encoded_outs, encoded_lens, embed, W_ih, W_hh, b_lstm,
           W_enc, W_pred, b_joint, W_out, b_out):
    f32 = jnp.float32
    enc_flat = jnp.transpose(encoded_outs, (1, 0, 2)).reshape(_T * _B, _DE)

    encp = pl.pallas_call(
        _proj_kernel,
        grid=(_T * _B // _MB,),
        in_specs=[
            pl.BlockSpec((_MB, _DE), lambda i: (i, 0)),
            pl.BlockSpec((_DE, _DJ), lambda i: (0, 0)),
            pl.BlockSpec((1, _DJ), lambda i: (0, 0)),
        ],
        out_specs=pl.BlockSpec((_MB, _DJ), lambda i: (i, 0)),
        out_shape=jax.ShapeDtypeStruct((_T * _B, _DJ), f32),
    )(enc_flat, W_enc, b_joint[None, :])
    encp = encp.reshape(_T, _B, _DJ)

    lens_b = jnp.broadcast_to(encoded_lens.astype(jnp.int32)[:, None],
                              (_B, 128))
    wih = [W_ih[:, k * _DP:(k + 1) * _DP] for k in range(4)]
    whh = [W_hh[:, k * _DP:(k + 1) * _DP] for k in range(4)]
    bls = [b_lstm[None, k * _DP:(k + 1) * _DP] for k in range(4)]

    labels, scores = pl.pallas_call(
        _decode_kernel,
        grid=(1,),
        in_specs=[
            _full((_T, _B, _DJ)),
            _full((_B, 128)),
            _full((_V, _DP)),
            _full((_DP, _DP)), _full((_DP, _DP)),
            _full((_DP, _DP)), _full((_DP, _DP)),
            _full((_DP, _DP)), _full((_DP, _DP)),
            _full((_DP, _DP)), _full((_DP, _DP)),
            _full((1, _DP)), _full((1, _DP)),
            _full((1, _DP)), _full((1, _DP)),
            _full((_DP, _DJ)),
            _full((_DJ, _V)),
            _full((1, _V)),
        ],
        out_specs=[
            _full((_B, _T)),
            _full((_B, _T)),
        ],
        out_shape=[
            jax.ShapeDtypeStruct((_B, _T), jnp.int32),
            jax.ShapeDtypeStruct((_B, _T), f32),
        ],
        compiler_params=pltpu.CompilerParams(
            dimension_semantics=("arbitrary",)),
    )(encp, lens_b, embed,
      wih[0], wih[1], wih[2], wih[3],
      whh[0], whh[1], whh[2], whh[3],
      bls[0], bls[1], bls[2], bls[3],
      W_pred, W_out, b_out[None, :])
    return labels, scores
